# Initial kernel scaffold; baseline (speedup 1.0000x reference)
import jax
import jax.numpy as jnp
from jax.experimental import pallas as pl


def kernel(x, edge_index, edge_attr, W_msg, W_edge, W_self, b, bn_gamma, bn_beta, prelu_a, W_res, rbn_gamma, rbn_beta):
    raise NotImplementedError("write your pallas kernel here")


# speedup vs baseline: 3.7733x; 3.7733x over previous
import functools

import jax
import jax.numpy as jnp
from jax import lax
from jax.experimental import pallas as pl
from jax.experimental.pallas import tpu as pltpu
from jax.experimental.pallas import tpu_sc as plsc

N_NODES = 10000
N_EDGES = 160000
D_IN = 256
D_HALF = 128
D_OUT = 256
D_EDGE = 16
EA_PAD = 128

NC = 2
NS = 16
B = 128
EP = 163840
K_X = EP // NS // B
K_E = EP // (NC * NS) // B
N_JUNK = 240
N_ACC = N_NODES + N_JUNK
ACC_ROWS_PER_TILE = N_ACC // NS

_EPS = 1e-5


def _sc_aggregate_x(x0, x1, src, dst):
    mesh = plsc.VectorSubcoreMesh(core_axis_name="c", subcore_axis_name="s")
    f32 = jnp.float32

    @functools.partial(
        pl.kernel,
        mesh=mesh,
        out_type=[
            jax.ShapeDtypeStruct((N_ACC, D_HALF), f32),
            jax.ShapeDtypeStruct((N_ACC, D_HALF), f32),
        ],
        scratch_types=[
            pltpu.VMEM((K_X, B), jnp.int32),
            pltpu.VMEM((K_X, B), jnp.int32),
            pltpu.VMEM((B, D_HALF), f32),
            pltpu.VMEM_SHARED((N_ACC, D_HALF), f32),
            pltpu.SemaphoreType.DMA,
        ],
    )
    def aggx_kernel(x0_hbm, x1_hbm, src_hbm, dst_hbm, outx0_hbm, outx1_hbm,
                    srcv, dstv, gbuf, accx, sem):
        c = lax.axis_index("c")
        s = lax.axis_index("s")

        zeros16 = jnp.zeros((16,), f32)

        @pl.loop(0, B)
        def _(r):
            @pl.loop(0, D_HALF, step=16)
            def _(q):
                gbuf[r, pl.ds(q, 16)] = zeros16

        @pl.loop(0, ACC_ROWS_PER_TILE, step=B)
        def _(r):
            pltpu.sync_copy(gbuf, accx.at[pl.ds(s * ACC_ROWS_PER_TILE + r, B)])

        plsc.subcore_barrier()

        pltpu.sync_copy(src_hbm.at[2 * s], srcv.at[pl.ds(0, K_E)])
        pltpu.sync_copy(src_hbm.at[2 * s + 1], srcv.at[pl.ds(K_E, K_E)])
        pltpu.sync_copy(dst_hbm.at[2 * s], dstv.at[pl.ds(0, K_E)])
        pltpu.sync_copy(dst_hbm.at[2 * s + 1], dstv.at[pl.ds(K_E, K_E)])

        def x_loop(xh_hbm):
            @pl.loop(0, K_X)
            def _(j):
                pltpu.async_copy(xh_hbm.at[srcv.at[j]], gbuf, sem).wait()
                pltpu.sync_copy(gbuf, accx.at[dstv.at[j]], add=True)

        @pl.when(c == 0)
        def _():
            x_loop(x0_hbm)

        @pl.when(c == 1)
        def _():
            x_loop(x1_hbm)

        plsc.subcore_barrier()

        rbase = s * ACC_ROWS_PER_TILE

        @pl.when(c == 0)
        def _():
            pltpu.sync_copy(accx.at[pl.ds(rbase, ACC_ROWS_PER_TILE)],
                            outx0_hbm.at[pl.ds(rbase, ACC_ROWS_PER_TILE)])

        @pl.when(c == 1)
        def _():
            pltpu.sync_copy(accx.at[pl.ds(rbase, ACC_ROWS_PER_TILE)],
                            outx1_hbm.at[pl.ds(rbase, ACC_ROWS_PER_TILE)])

    return aggx_kernel(x0, x1, src, dst)


def _sc_aggregate_e(dst, ea):
    mesh = plsc.VectorSubcoreMesh(core_axis_name="c", subcore_axis_name="s")
    f32 = jnp.float32

    @functools.partial(
        pl.kernel,
        mesh=mesh,
        out_type=[
            jax.ShapeDtypeStruct((N_ACC, EA_PAD), f32),
            jax.ShapeDtypeStruct((N_ACC, EA_PAD), f32),
        ],
        scratch_types=[
            pltpu.VMEM((K_E, B), jnp.int32),
            pltpu.VMEM((B, EA_PAD), f32),
            pltpu.VMEM_SHARED((N_ACC, EA_PAD), f32),
            pltpu.SemaphoreType.DMA,
        ],
    )
    def agge_kernel(dst_hbm, ea_hbm, oute0_hbm, oute1_hbm,
                    dstev, ebuf, acce, sem):
        c = lax.axis_index("c")
        s = lax.axis_index("s")
        w = s * NC + c

        zeros16 = jnp.zeros((16,), f32)

        @pl.loop(0, B)
        def _(r):
            @pl.loop(0, EA_PAD, step=16)
            def _(q):
                ebuf[r, pl.ds(q, 16)] = zeros16

        @pl.loop(0, ACC_ROWS_PER_TILE, step=B)
        def _(r):
            pltpu.sync_copy(ebuf, acce.at[pl.ds(s * ACC_ROWS_PER_TILE + r, B)])

        plsc.subcore_barrier()

        pltpu.sync_copy(dst_hbm.at[w], dstev)

        @pl.loop(0, K_E)
        def _(j):
            pltpu.sync_copy(ea_hbm.at[w * K_E + j], ebuf)
            pltpu.sync_copy(ebuf, acce.at[dstev.at[j]], add=True)

        plsc.subcore_barrier()

        rbase = s * ACC_ROWS_PER_TILE

        @pl.when(c == 0)
        def _():
            pltpu.sync_copy(acce.at[pl.ds(rbase, ACC_ROWS_PER_TILE)],
                            oute0_hbm.at[pl.ds(rbase, ACC_ROWS_PER_TILE)])

        @pl.when(c == 1)
        def _():
            pltpu.sync_copy(acce.at[pl.ds(rbase, ACC_ROWS_PER_TILE)],
                            oute1_hbm.at[pl.ds(rbase, ACC_ROWS_PER_TILE)])

    return agge_kernel(dst, ea)


NB = 10
BLK = N_NODES // NB


def _dot(a, bm):
    return lax.dot_general(a, bm, (((1,), (0,)), ((), ())),
                           precision=lax.Precision.HIGHEST,
                           preferred_element_type=jnp.float32)


def _tc_a_body(x_ref, ax0_ref, ax1_ref, ae0_ref, ae1_ref,
               wm_ref, we_ref, ws_ref, b_ref, wr_ref,
               hpre_ref, respre_ref, stats_ref):
    i = pl.program_id(0)
    xb = x_ref[...]
    r = _dot(xb, wr_ref[...])
    ae = ae0_ref[...] + ae1_ref[...]
    deg = jnp.maximum(ae[:, D_EDGE:D_EDGE + 1], 1.0)
    ssum = (_dot(ax0_ref[...], wm_ref[0:D_HALF, :])
            + _dot(ax1_ref[...], wm_ref[D_HALF:D_IN, :])
            + _dot(ae[:, 0:D_EDGE], we_ref[...]))
    h = ssum / deg + _dot(xb, ws_ref[...]) + b_ref[...]
    hpre_ref[...] = h
    respre_ref[...] = r

    st = jnp.concatenate(
        [jnp.sum(h, axis=0, keepdims=True),
         jnp.sum(h * h, axis=0, keepdims=True),
         jnp.sum(r, axis=0, keepdims=True),
         jnp.sum(r * r, axis=0, keepdims=True),
         jnp.zeros((4, D_OUT), jnp.float32)], axis=0)

    @pl.when(i == 0)
    def _():
        stats_ref[...] = st

    @pl.when(i > 0)
    def _():
        stats_ref[...] += st


def _tc_a(x, ax0, ax1, ae0, ae1, wm, we, ws, b2d, wr):
    blk = lambda cols: pl.BlockSpec((BLK, cols), lambda i: (i, 0))
    full = lambda a, c: pl.BlockSpec((a, c), lambda i: (0, 0))
    return pl.pallas_call(
        _tc_a_body,
        grid=(NB,),
        in_specs=[
            blk(D_IN), blk(D_HALF), blk(D_HALF), blk(EA_PAD), blk(EA_PAD),
            full(D_IN, D_OUT), full(D_EDGE, D_OUT), full(D_IN, D_OUT),
            full(1, D_OUT), full(D_IN, D_OUT),
        ],
        out_specs=[blk(D_OUT), blk(D_OUT),
                   pl.BlockSpec((8, D_OUT), lambda i: (0, 0))],
        out_shape=[
            jax.ShapeDtypeStruct((N_NODES, D_OUT), jnp.float32),
            jax.ShapeDtypeStruct((N_NODES, D_OUT), jnp.float32),
            jax.ShapeDtypeStruct((8, D_OUT), jnp.float32),
        ],
    )(x, ax0, ax1, ae0, ae1, wm, we, ws, b2d, wr)


def _tc_b_body(hpre_ref, respre_ref, stats_ref, g1_ref, b1_ref,
               g2_ref, b2_ref, a_ref, out_ref):
    st = stats_ref[...]
    inv_n = 1.0 / N_NODES
    mu_h = st[0:1, :] * inv_n
    var_h = st[1:2, :] * inv_n - mu_h * mu_h
    mu_r = st[2:3, :] * inv_n
    var_r = st[3:4, :] * inv_n - mu_r * mu_r

    h = hpre_ref[...]
    hn = (h - mu_h) * lax.rsqrt(var_h + _EPS) * g1_ref[...] + b1_ref[...]
    a = a_ref[0, 0]
    hn = jnp.where(hn > 0, hn, a * hn)

    r = respre_ref[...]
    rn = (r - mu_r) * lax.rsqrt(var_r + _EPS) * g2_ref[...] + b2_ref[...]
    out_ref[...] = hn + rn


def _tc_b(hpre, respre, stats, g1, b1, g2, b2, a2d):
    blk = pl.BlockSpec((BLK, D_OUT), lambda i: (i, 0))
    full = lambda r, c: pl.BlockSpec((r, c), lambda i: (0, 0))
    return pl.pallas_call(
        _tc_b_body,
        grid=(NB,),
        in_specs=[blk, blk, full(8, D_OUT), full(1, D_OUT), full(1, D_OUT),
                  full(1, D_OUT), full(1, D_OUT), full(1, 1)],
        out_specs=blk,
        out_shape=jax.ShapeDtypeStruct((N_NODES, D_OUT), jnp.float32),
    )(hpre, respre, stats, g1, b1, g2, b2, a2d)


def kernel(x, edge_index, edge_attr, W_msg, W_edge, W_self, b,
           bn_gamma, bn_beta, prelu_a, W_res, rbn_gamma, rbn_beta):
    i32 = jnp.int32
    src = edge_index[0].astype(i32)
    dst = edge_index[1].astype(i32)

    npad = EP - N_EDGES
    pad_ar = jnp.arange(npad, dtype=i32)
    src_p = jnp.concatenate([src, pad_ar % N_NODES])
    dst_p = jnp.concatenate([dst, N_NODES + pad_ar % N_JUNK])

    src3 = src_p.reshape(NC * NS, K_E, B)
    dst3 = dst_p.reshape(NC * NS, K_E, B)

    ea = jnp.concatenate(
        [edge_attr,
         jnp.ones((N_EDGES, 1), jnp.float32),
         jnp.zeros((N_EDGES, EA_PAD - D_EDGE - 1), jnp.float32)], axis=1)
    ea = jnp.concatenate([ea, jnp.zeros((npad, EA_PAD), jnp.float32)], axis=0)
    ea = ea.reshape(NC * NS * K_E, B, EA_PAD)

    x0 = x[:, :D_HALF]
    x1 = x[:, D_HALF:]

    ax0, ax1 = _sc_aggregate_x(x0, x1, src3, dst3)
    ae0, ae1 = _sc_aggregate_e(dst3, ea)

    b2d = b.reshape(1, D_OUT)
    hpre, respre, stats = _tc_a(x, ax0, ax1, ae0, ae1,
                                W_msg, W_edge, W_self, b2d, W_res)

    out = _tc_b(hpre, respre, stats,
                bn_gamma.reshape(1, D_OUT), bn_beta.reshape(1, D_OUT),
                rbn_gamma.reshape(1, D_OUT), rbn_beta.reshape(1, D_OUT),
                jnp.asarray(prelu_a, jnp.float32).reshape(1, 1))
    return out


# speedup vs baseline: 4.3981x; 1.1656x over previous
import functools

import jax
import jax.numpy as jnp
from jax import lax
from jax.experimental import pallas as pl
from jax.experimental.pallas import tpu as pltpu
from jax.experimental.pallas import tpu_sc as plsc

N_NODES = 10000
N_EDGES = 160000
D_IN = 256
D_HALF = 128
D_OUT = 256
D_EDGE = 16
EA_PAD = 128

NC = 2
NS = 16
B = 128
EP = 163840
K_X = EP // NS // B
K_E = EP // (NC * NS) // B
N_JUNK = 240
N_ACC = N_NODES + N_JUNK
ACC_ROWS_PER_TILE = N_ACC // NS

_EPS = 1e-5


def _sc_aggregate_x(x0, x1, src, dst):
    mesh = plsc.VectorSubcoreMesh(core_axis_name="c", subcore_axis_name="s")
    f32 = jnp.float32

    @functools.partial(
        pl.kernel,
        mesh=mesh,
        out_type=[
            jax.ShapeDtypeStruct((N_ACC, D_HALF), f32),
            jax.ShapeDtypeStruct((N_ACC, D_HALF), f32),
        ],
        scratch_types=[
            pltpu.VMEM((K_E, B), jnp.int32),
            pltpu.VMEM((K_E, B), jnp.int32),
            pltpu.VMEM((B, D_HALF), f32),
            pltpu.VMEM((B, D_HALF), f32),
            pltpu.VMEM_SHARED((N_ACC, D_HALF), f32),
            pltpu.SemaphoreType.DMA,
            pltpu.SemaphoreType.DMA,
        ],
    )
    def aggx_kernel(x0_hbm, x1_hbm, src_hbm, dst_hbm, outx0_hbm, outx1_hbm,
                    srcv, dstv, gbuf, gbuf1, accx, sem, sem1):
        c = lax.axis_index("c")
        s = lax.axis_index("s")

        zeros16 = jnp.zeros((16,), f32)

        @pl.loop(0, B)
        def _(r):
            @pl.loop(0, D_HALF, step=16)
            def _(q):
                gbuf[r, pl.ds(q, 16)] = zeros16

        @pl.loop(0, ACC_ROWS_PER_TILE, step=B)
        def _(r):
            pltpu.sync_copy(gbuf, accx.at[pl.ds(s * ACC_ROWS_PER_TILE + r, B)])

        plsc.subcore_barrier()

        def x_loop(xh_hbm):
            for p in range(2):
                pltpu.sync_copy(src_hbm.at[2 * s + p], srcv)
                pltpu.sync_copy(dst_hbm.at[2 * s + p], dstv)
                pltpu.async_copy(xh_hbm.at[srcv.at[0]], gbuf, sem)

                @pl.loop(0, K_E - 2, step=2)
                def _(j):
                    pltpu.make_async_copy(
                        xh_hbm.at[srcv.at[j]], gbuf, sem).wait()
                    pltpu.async_copy(xh_hbm.at[srcv.at[j + 1]], gbuf1, sem1)
                    pltpu.sync_copy(gbuf, accx.at[dstv.at[j]], add=True)
                    pltpu.make_async_copy(
                        xh_hbm.at[srcv.at[j + 1]], gbuf1, sem1).wait()
                    pltpu.async_copy(xh_hbm.at[srcv.at[j + 2]], gbuf, sem)
                    pltpu.sync_copy(gbuf1, accx.at[dstv.at[j + 1]], add=True)

                jt = K_E - 2
                pltpu.make_async_copy(xh_hbm.at[srcv.at[jt]], gbuf, sem).wait()
                pltpu.async_copy(xh_hbm.at[srcv.at[jt + 1]], gbuf1, sem1)
                pltpu.sync_copy(gbuf, accx.at[dstv.at[jt]], add=True)
                pltpu.make_async_copy(
                    xh_hbm.at[srcv.at[jt + 1]], gbuf1, sem1).wait()
                pltpu.sync_copy(gbuf1, accx.at[dstv.at[jt + 1]], add=True)

        @pl.when(c == 0)
        def _():
            x_loop(x0_hbm)

        @pl.when(c == 1)
        def _():
            x_loop(x1_hbm)

        plsc.subcore_barrier()

        rbase = s * ACC_ROWS_PER_TILE

        @pl.when(c == 0)
        def _():
            pltpu.sync_copy(accx.at[pl.ds(rbase, ACC_ROWS_PER_TILE)],
                            outx0_hbm.at[pl.ds(rbase, ACC_ROWS_PER_TILE)])

        @pl.when(c == 1)
        def _():
            pltpu.sync_copy(accx.at[pl.ds(rbase, ACC_ROWS_PER_TILE)],
                            outx1_hbm.at[pl.ds(rbase, ACC_ROWS_PER_TILE)])

    return aggx_kernel(x0, x1, src, dst)


def _sc_aggregate_e(dst, ea):
    mesh = plsc.VectorSubcoreMesh(core_axis_name="c", subcore_axis_name="s")
    f32 = jnp.float32

    @functools.partial(
        pl.kernel,
        mesh=mesh,
        out_type=[
            jax.ShapeDtypeStruct((N_ACC, EA_PAD), f32),
            jax.ShapeDtypeStruct((N_ACC, EA_PAD), f32),
        ],
        scratch_types=[
            pltpu.VMEM((K_E, B), jnp.int32),
            pltpu.VMEM((B, EA_PAD), f32),
            pltpu.VMEM((B, EA_PAD), f32),
            pltpu.VMEM_SHARED((N_ACC, EA_PAD), f32),
            pltpu.SemaphoreType.DMA,
            pltpu.SemaphoreType.DMA,
        ],
    )
    def agge_kernel(dst_hbm, ea_hbm, oute0_hbm, oute1_hbm,
                    dstev, ebuf, ebuf1, acce, sem, sem1):
        c = lax.axis_index("c")
        s = lax.axis_index("s")
        w = s * NC + c

        zeros16 = jnp.zeros((16,), f32)

        @pl.loop(0, B)
        def _(r):
            @pl.loop(0, EA_PAD, step=16)
            def _(q):
                ebuf[r, pl.ds(q, 16)] = zeros16

        @pl.loop(0, ACC_ROWS_PER_TILE, step=B)
        def _(r):
            pltpu.sync_copy(ebuf, acce.at[pl.ds(s * ACC_ROWS_PER_TILE + r, B)])

        plsc.subcore_barrier()

        pltpu.sync_copy(dst_hbm.at[w], dstev)

        pltpu.async_copy(ea_hbm.at[w * K_E], ebuf, sem)

        @pl.loop(0, K_E - 2, step=2)
        def _(j):
            pltpu.make_async_copy(ea_hbm.at[w * K_E + j], ebuf, sem).wait()
            pltpu.async_copy(ea_hbm.at[w * K_E + j + 1], ebuf1, sem1)
            pltpu.sync_copy(ebuf, acce.at[dstev.at[j]], add=True)
            pltpu.make_async_copy(
                ea_hbm.at[w * K_E + j + 1], ebuf1, sem1).wait()
            pltpu.async_copy(ea_hbm.at[w * K_E + j + 2], ebuf, sem)
            pltpu.sync_copy(ebuf1, acce.at[dstev.at[j + 1]], add=True)

        jt = K_E - 2
        pltpu.make_async_copy(ea_hbm.at[w * K_E + jt], ebuf, sem).wait()
        pltpu.async_copy(ea_hbm.at[w * K_E + jt + 1], ebuf1, sem1)
        pltpu.sync_copy(ebuf, acce.at[dstev.at[jt]], add=True)
        pltpu.make_async_copy(
            ea_hbm.at[w * K_E + jt + 1], ebuf1, sem1).wait()
        pltpu.sync_copy(ebuf1, acce.at[dstev.at[jt + 1]], add=True)

        plsc.subcore_barrier()

        rbase = s * ACC_ROWS_PER_TILE

        @pl.when(c == 0)
        def _():
            pltpu.sync_copy(acce.at[pl.ds(rbase, ACC_ROWS_PER_TILE)],
                            oute0_hbm.at[pl.ds(rbase, ACC_ROWS_PER_TILE)])

        @pl.when(c == 1)
        def _():
            pltpu.sync_copy(acce.at[pl.ds(rbase, ACC_ROWS_PER_TILE)],
                            oute1_hbm.at[pl.ds(rbase, ACC_ROWS_PER_TILE)])

    return agge_kernel(dst, ea)


NB = 10
BLK = N_NODES // NB


def _dot(a, bm):
    return lax.dot_general(a, bm, (((1,), (0,)), ((), ())),
                           precision=lax.Precision.HIGHEST,
                           preferred_element_type=jnp.float32)


def _tc_a_body(x_ref, ax0_ref, ax1_ref, ae0_ref, ae1_ref,
               wm_ref, we_ref, ws_ref, b_ref, wr_ref,
               hpre_ref, respre_ref, stats_ref):
    i = pl.program_id(0)
    xb = x_ref[...]
    r = _dot(xb, wr_ref[...])
    ae = ae0_ref[...] + ae1_ref[...]
    deg = jnp.maximum(ae[:, D_EDGE:D_EDGE + 1], 1.0)
    ssum = (_dot(ax0_ref[...], wm_ref[0:D_HALF, :])
            + _dot(ax1_ref[...], wm_ref[D_HALF:D_IN, :])
            + _dot(ae[:, 0:D_EDGE], we_ref[...]))
    h = ssum / deg + _dot(xb, ws_ref[...]) + b_ref[...]
    hpre_ref[...] = h
    respre_ref[...] = r

    st = jnp.concatenate(
        [jnp.sum(h, axis=0, keepdims=True),
         jnp.sum(h * h, axis=0, keepdims=True),
         jnp.sum(r, axis=0, keepdims=True),
         jnp.sum(r * r, axis=0, keepdims=True),
         jnp.zeros((4, D_OUT), jnp.float32)], axis=0)

    @pl.when(i == 0)
    def _():
        stats_ref[...] = st

    @pl.when(i > 0)
    def _():
        stats_ref[...] += st


def _tc_a(x, ax0, ax1, ae0, ae1, wm, we, ws, b2d, wr):
    blk = lambda cols: pl.BlockSpec((BLK, cols), lambda i: (i, 0))
    full = lambda a, c: pl.BlockSpec((a, c), lambda i: (0, 0))
    return pl.pallas_call(
        _tc_a_body,
        grid=(NB,),
        in_specs=[
            blk(D_IN), blk(D_HALF), blk(D_HALF), blk(EA_PAD), blk(EA_PAD),
            full(D_IN, D_OUT), full(D_EDGE, D_OUT), full(D_IN, D_OUT),
            full(1, D_OUT), full(D_IN, D_OUT),
        ],
        out_specs=[blk(D_OUT), blk(D_OUT),
                   pl.BlockSpec((8, D_OUT), lambda i: (0, 0))],
        out_shape=[
            jax.ShapeDtypeStruct((N_NODES, D_OUT), jnp.float32),
            jax.ShapeDtypeStruct((N_NODES, D_OUT), jnp.float32),
            jax.ShapeDtypeStruct((8, D_OUT), jnp.float32),
        ],
    )(x, ax0, ax1, ae0, ae1, wm, we, ws, b2d, wr)


def _tc_b_body(hpre_ref, respre_ref, stats_ref, g1_ref, b1_ref,
               g2_ref, b2_ref, a_ref, out_ref):
    st = stats_ref[...]
    inv_n = 1.0 / N_NODES
    mu_h = st[0:1, :] * inv_n
    var_h = st[1:2, :] * inv_n - mu_h * mu_h
    mu_r = st[2:3, :] * inv_n
    var_r = st[3:4, :] * inv_n - mu_r * mu_r

    h = hpre_ref[...]
    hn = (h - mu_h) * lax.rsqrt(var_h + _EPS) * g1_ref[...] + b1_ref[...]
    a = a_ref[0, 0]
    hn = jnp.where(hn > 0, hn, a * hn)

    r = respre_ref[...]
    rn = (r - mu_r) * lax.rsqrt(var_r + _EPS) * g2_ref[...] + b2_ref[...]
    out_ref[...] = hn + rn


def _tc_b(hpre, respre, stats, g1, b1, g2, b2, a2d):
    blk = pl.BlockSpec((BLK, D_OUT), lambda i: (i, 0))
    full = lambda r, c: pl.BlockSpec((r, c), lambda i: (0, 0))
    return pl.pallas_call(
        _tc_b_body,
        grid=(NB,),
        in_specs=[blk, blk, full(8, D_OUT), full(1, D_OUT), full(1, D_OUT),
                  full(1, D_OUT), full(1, D_OUT), full(1, 1)],
        out_specs=blk,
        out_shape=jax.ShapeDtypeStruct((N_NODES, D_OUT), jnp.float32),
    )(hpre, respre, stats, g1, b1, g2, b2, a2d)


def kernel(x, edge_index, edge_attr, W_msg, W_edge, W_self, b,
           bn_gamma, bn_beta, prelu_a, W_res, rbn_gamma, rbn_beta):
    i32 = jnp.int32
    src = edge_index[0].astype(i32)
    dst = edge_index[1].astype(i32)

    npad = EP - N_EDGES
    pad_ar = jnp.arange(npad, dtype=i32)
    src_p = jnp.concatenate([src, pad_ar % N_NODES])
    dst_p = jnp.concatenate([dst, N_NODES + pad_ar % N_JUNK])

    src3 = src_p.reshape(NC * NS, K_E, B)
    dst3 = dst_p.reshape(NC * NS, K_E, B)

    ea = jnp.concatenate(
        [edge_attr,
         jnp.ones((N_EDGES, 1), jnp.float32),
         jnp.zeros((N_EDGES, EA_PAD - D_EDGE - 1), jnp.float32)], axis=1)
    ea = jnp.concatenate([ea, jnp.zeros((npad, EA_PAD), jnp.float32)], axis=0)
    ea = ea.reshape(NC * NS * K_E, B, EA_PAD)

    x0 = x[:, :D_HALF]
    x1 = x[:, D_HALF:]

    ax0, ax1 = _sc_aggregate_x(x0, x1, src3, dst3)
    ae0, ae1 = _sc_aggregate_e(dst3, ea)

    b2d = b.reshape(1, D_OUT)
    hpre, respre, stats = _tc_a(x, ax0, ax1, ae0, ae1,
                                W_msg, W_edge, W_self, b2d, W_res)

    out = _tc_b(hpre, respre, stats,
                bn_gamma.reshape(1, D_OUT), bn_beta.reshape(1, D_OUT),
                rbn_gamma.reshape(1, D_OUT), rbn_beta.reshape(1, D_OUT),
                jnp.asarray(prelu_a, jnp.float32).reshape(1, 1))
    return out


# speedup vs baseline: 4.8162x; 1.0951x over previous
import functools

import jax
import jax.numpy as jnp
from jax import lax
from jax.experimental import pallas as pl
from jax.experimental.pallas import tpu as pltpu
from jax.experimental.pallas import tpu_sc as plsc

N_NODES = 10000
N_EDGES = 160000
D_IN = 256
D_HALF = 128
D_OUT = 256
D_EDGE = 16
EA_PAD = 128

NC = 2
NS = 16
B = 128
EP = 163840
K_X = EP // NS // B
K_E = EP // (NC * NS) // B
N_JUNK = 240
N_ACC = N_NODES + N_JUNK
ACC_ROWS_PER_TILE = N_ACC // NS

_EPS = 1e-5


def _sc_aggregate_x(x0, x1, src, dst):
    mesh = plsc.VectorSubcoreMesh(core_axis_name="c", subcore_axis_name="s")
    f32 = jnp.float32

    @functools.partial(
        pl.kernel,
        mesh=mesh,
        out_type=[
            jax.ShapeDtypeStruct((N_ACC, D_HALF), f32),
            jax.ShapeDtypeStruct((N_ACC, D_HALF), f32),
        ],
        scratch_types=[
            pltpu.VMEM((K_E, B), jnp.int32),
            pltpu.VMEM((K_E, B), jnp.int32),
            pltpu.VMEM((B, D_HALF), f32),
            pltpu.VMEM((B, D_HALF), f32),
            pltpu.VMEM_SHARED((N_ACC, D_HALF), f32),
            pltpu.SemaphoreType.DMA,
            pltpu.SemaphoreType.DMA,
        ],
    )
    def aggx_kernel(x0_hbm, x1_hbm, src_hbm, dst_hbm, outx0_hbm, outx1_hbm,
                    srcv, dstv, gbuf, gbuf1, accx, sem, sem1):
        c = lax.axis_index("c")
        s = lax.axis_index("s")

        zeros16 = jnp.zeros((16,), f32)

        @pl.loop(0, B)
        def _(r):
            @pl.loop(0, D_HALF, step=16)
            def _(q):
                gbuf[r, pl.ds(q, 16)] = zeros16

        @pl.loop(0, ACC_ROWS_PER_TILE, step=B)
        def _(r):
            pltpu.sync_copy(gbuf, accx.at[pl.ds(s * ACC_ROWS_PER_TILE + r, B)])

        plsc.subcore_barrier()

        def x_loop(xh_hbm):
            for p in range(2):
                pltpu.sync_copy(src_hbm.at[2 * s + p], srcv)
                pltpu.sync_copy(dst_hbm.at[2 * s + p], dstv)
                pltpu.async_copy(xh_hbm.at[srcv.at[0]], gbuf, sem)

                @pl.loop(0, K_E - 2, step=2)
                def _(j):
                    pltpu.make_async_copy(
                        xh_hbm.at[srcv.at[j]], gbuf, sem).wait()
                    pltpu.async_copy(xh_hbm.at[srcv.at[j + 1]], gbuf1, sem1)
                    pltpu.sync_copy(gbuf, accx.at[dstv.at[j]], add=True)
                    pltpu.make_async_copy(
                        xh_hbm.at[srcv.at[j + 1]], gbuf1, sem1).wait()
                    pltpu.async_copy(xh_hbm.at[srcv.at[j + 2]], gbuf, sem)
                    pltpu.sync_copy(gbuf1, accx.at[dstv.at[j + 1]], add=True)

                jt = K_E - 2
                pltpu.make_async_copy(xh_hbm.at[srcv.at[jt]], gbuf, sem).wait()
                pltpu.async_copy(xh_hbm.at[srcv.at[jt + 1]], gbuf1, sem1)
                pltpu.sync_copy(gbuf, accx.at[dstv.at[jt]], add=True)
                pltpu.make_async_copy(
                    xh_hbm.at[srcv.at[jt + 1]], gbuf1, sem1).wait()
                pltpu.sync_copy(gbuf1, accx.at[dstv.at[jt + 1]], add=True)

        @pl.when(c == 0)
        def _():
            x_loop(x0_hbm)

        @pl.when(c == 1)
        def _():
            x_loop(x1_hbm)

        plsc.subcore_barrier()

        rbase = s * ACC_ROWS_PER_TILE

        @pl.when(c == 0)
        def _():
            pltpu.sync_copy(accx.at[pl.ds(rbase, ACC_ROWS_PER_TILE)],
                            outx0_hbm.at[pl.ds(rbase, ACC_ROWS_PER_TILE)])

        @pl.when(c == 1)
        def _():
            pltpu.sync_copy(accx.at[pl.ds(rbase, ACC_ROWS_PER_TILE)],
                            outx1_hbm.at[pl.ds(rbase, ACC_ROWS_PER_TILE)])

    return aggx_kernel(x0, x1, src, dst)


def _sc_aggregate_e(dst, ea):
    mesh = plsc.VectorSubcoreMesh(core_axis_name="c", subcore_axis_name="s")
    f32 = jnp.float32

    @functools.partial(
        pl.kernel,
        mesh=mesh,
        out_type=jax.ShapeDtypeStruct((NC, N_ACC, EA_PAD), f32),
        scratch_types=[
            pltpu.VMEM((K_E, B), jnp.int32),
            pltpu.VMEM((D_EDGE, B), f32),
            pltpu.VMEM((D_EDGE, B), f32),
            pltpu.VMEM((B, EA_PAD), f32),
            pltpu.VMEM_SHARED((N_ACC, EA_PAD), f32),
            pltpu.SemaphoreType.DMA,
            pltpu.SemaphoreType.DMA,
        ],
    )
    def agge_kernel(dst_hbm, ea_hbm, oute_hbm,
                    dstev, rbuf0, rbuf1, ebuf0, acce, semr0, semr1):
        c = lax.axis_index("c")
        s = lax.axis_index("s")
        w = s * NC + c

        zeros16 = jnp.zeros((16,), f32)
        onehot = jnp.where(lax.iota(jnp.int32, 16) == 0,
                           jnp.float32(1), jnp.float32(0))

        @pl.loop(0, B)
        def _(r):
            @pl.loop(0, EA_PAD, step=16)
            def _(q):
                ebuf0[r, pl.ds(q, 16)] = zeros16

        @pl.loop(0, ACC_ROWS_PER_TILE, step=B)
        def _(r):
            pltpu.sync_copy(ebuf0, acce.at[pl.ds(s * ACC_ROWS_PER_TILE + r, B)])

        plsc.subcore_barrier()

        @pl.loop(0, B)
        def _(r):
            ebuf0[r, pl.ds(D_EDGE, 16)] = onehot

        pltpu.sync_copy(dst_hbm.at[w], dstev)

        def read(j, rb, semr):
            pltpu.async_copy(ea_hbm.at[w * K_E + j], rb, semr)

        def wait_read(j, rb, semr):
            pltpu.make_async_copy(ea_hbm.at[w * K_E + j], rb, semr).wait()

        def expand(rb):
            @pl.loop(0, D_EDGE)
            def _(rr):
                for k in range(8):
                    ebuf0[rr * 8 + k, pl.ds(0, D_EDGE)] = (
                        rb[rr, pl.ds(k * D_EDGE, D_EDGE)])

        def scat(j):
            pltpu.sync_copy(ebuf0, acce.at[dstev.at[j]], add=True)

        read(0, rbuf0, semr0)
        read(1, rbuf1, semr1)

        @pl.loop(0, K_E - 2, step=2)
        def _(j):
            wait_read(j, rbuf0, semr0)
            expand(rbuf0)
            read(j + 2, rbuf0, semr0)
            scat(j)
            wait_read(j + 1, rbuf1, semr1)
            expand(rbuf1)
            read(j + 3, rbuf1, semr1)
            scat(j + 1)

        jt = K_E - 2
        wait_read(jt, rbuf0, semr0)
        expand(rbuf0)
        scat(jt)
        wait_read(jt + 1, rbuf1, semr1)
        expand(rbuf1)
        scat(jt + 1)

        plsc.subcore_barrier()

        rbase = s * ACC_ROWS_PER_TILE
        pltpu.sync_copy(acce.at[pl.ds(rbase, ACC_ROWS_PER_TILE)],
                        oute_hbm.at[c, pl.ds(rbase, ACC_ROWS_PER_TILE)])

    return agge_kernel(dst, ea)


NB = 10
BLK = N_NODES // NB


def _dot(a, bm):
    return lax.dot_general(a, bm, (((1,), (0,)), ((), ())),
                           precision=lax.Precision.HIGHEST,
                           preferred_element_type=jnp.float32)


def _tc_a_body(x_ref, ax0_ref, ax1_ref, ae_ref,
               wm_ref, we_ref, ws_ref, b_ref, wr_ref,
               hpre_ref, respre_ref, stats_ref):
    i = pl.program_id(0)
    xb = x_ref[...]
    r = _dot(xb, wr_ref[...])
    ae = ae_ref[0] + ae_ref[1]
    deg = jnp.maximum(ae[:, D_EDGE:D_EDGE + 1], 1.0)
    ssum = (_dot(ax0_ref[...], wm_ref[0:D_HALF, :])
            + _dot(ax1_ref[...], wm_ref[D_HALF:D_IN, :])
            + _dot(ae[:, 0:D_EDGE], we_ref[...]))
    h = ssum / deg + _dot(xb, ws_ref[...]) + b_ref[...]
    hpre_ref[...] = h
    respre_ref[...] = r

    st = jnp.concatenate(
        [jnp.sum(h, axis=0, keepdims=True),
         jnp.sum(h * h, axis=0, keepdims=True),
         jnp.sum(r, axis=0, keepdims=True),
         jnp.sum(r * r, axis=0, keepdims=True),
         jnp.zeros((4, D_OUT), jnp.float32)], axis=0)

    @pl.when(i == 0)
    def _():
        stats_ref[...] = st

    @pl.when(i > 0)
    def _():
        stats_ref[...] += st


def _tc_a(x, ax0, ax1, ae, wm, we, ws, b2d, wr):
    blk = lambda cols: pl.BlockSpec((BLK, cols), lambda i: (i, 0))
    full = lambda a, c: pl.BlockSpec((a, c), lambda i: (0, 0))
    return pl.pallas_call(
        _tc_a_body,
        grid=(NB,),
        in_specs=[
            blk(D_IN), blk(D_HALF), blk(D_HALF),
            pl.BlockSpec((NC, BLK, EA_PAD), lambda i: (0, i, 0)),
            full(D_IN, D_OUT), full(D_EDGE, D_OUT), full(D_IN, D_OUT),
            full(1, D_OUT), full(D_IN, D_OUT),
        ],
        out_specs=[blk(D_OUT), blk(D_OUT),
                   pl.BlockSpec((8, D_OUT), lambda i: (0, 0))],
        out_shape=[
            jax.ShapeDtypeStruct((N_NODES, D_OUT), jnp.float32),
            jax.ShapeDtypeStruct((N_NODES, D_OUT), jnp.float32),
            jax.ShapeDtypeStruct((8, D_OUT), jnp.float32),
        ],
    )(x, ax0, ax1, ae, wm, we, ws, b2d, wr)


def _tc_b_body(hpre_ref, respre_ref, stats_ref, g1_ref, b1_ref,
               g2_ref, b2_ref, a_ref, out_ref):
    st = stats_ref[...]
    inv_n = 1.0 / N_NODES
    mu_h = st[0:1, :] * inv_n
    var_h = st[1:2, :] * inv_n - mu_h * mu_h
    mu_r = st[2:3, :] * inv_n
    var_r = st[3:4, :] * inv_n - mu_r * mu_r

    h = hpre_ref[...]
    hn = (h - mu_h) * lax.rsqrt(var_h + _EPS) * g1_ref[...] + b1_ref[...]
    a = a_ref[0, 0]
    hn = jnp.where(hn > 0, hn, a * hn)

    r = respre_ref[...]
    rn = (r - mu_r) * lax.rsqrt(var_r + _EPS) * g2_ref[...] + b2_ref[...]
    out_ref[...] = hn + rn


def _tc_b(hpre, respre, stats, g1, b1, g2, b2, a2d):
    blk = pl.BlockSpec((BLK, D_OUT), lambda i: (i, 0))
    full = lambda r, c: pl.BlockSpec((r, c), lambda i: (0, 0))
    return pl.pallas_call(
        _tc_b_body,
        grid=(NB,),
        in_specs=[blk, blk, full(8, D_OUT), full(1, D_OUT), full(1, D_OUT),
                  full(1, D_OUT), full(1, D_OUT), full(1, 1)],
        out_specs=blk,
        out_shape=jax.ShapeDtypeStruct((N_NODES, D_OUT), jnp.float32),
    )(hpre, respre, stats, g1, b1, g2, b2, a2d)


def kernel(x, edge_index, edge_attr, W_msg, W_edge, W_self, b,
           bn_gamma, bn_beta, prelu_a, W_res, rbn_gamma, rbn_beta):
    i32 = jnp.int32
    src = edge_index[0].astype(i32)
    dst = edge_index[1].astype(i32)

    npad = EP - N_EDGES
    pad_ar = jnp.arange(npad, dtype=i32)
    src_p = jnp.concatenate([src, pad_ar % N_NODES])
    dst_p = jnp.concatenate([dst, N_NODES + pad_ar % N_JUNK])

    src3 = src_p.reshape(NC * NS, K_E, B)
    dst3 = dst_p.reshape(NC * NS, K_E, B)

    ea = jnp.concatenate(
        [edge_attr, jnp.zeros((npad, D_EDGE), jnp.float32)], axis=0)
    ea = ea.reshape(NC * NS * K_E, D_EDGE, B)

    x0 = x[:, :D_HALF]
    x1 = x[:, D_HALF:]

    ax0, ax1 = _sc_aggregate_x(x0, x1, src3, dst3)
    ae = _sc_aggregate_e(dst3, ea)

    b2d = b.reshape(1, D_OUT)
    hpre, respre, stats = _tc_a(x, ax0, ax1, ae,
                                W_msg, W_edge, W_self, b2d, W_res)

    out = _tc_b(hpre, respre, stats,
                bn_gamma.reshape(1, D_OUT), bn_beta.reshape(1, D_OUT),
                rbn_gamma.reshape(1, D_OUT), rbn_beta.reshape(1, D_OUT),
                jnp.asarray(prelu_a, jnp.float32).reshape(1, 1))
    return out


# speedup vs baseline: 5.0185x; 1.0420x over previous
import functools

import jax
import jax.numpy as jnp
from jax import lax
from jax.experimental import pallas as pl
from jax.experimental.pallas import tpu as pltpu
from jax.experimental.pallas import tpu_sc as plsc

N_NODES = 10000
N_EDGES = 160000
D_IN = 256
D_HALF = 128
D_OUT = 256
D_EDGE = 16
EA_PAD = 128

NC = 2
NS = 16
B = 128
EP = 163840
K_X = EP // NS // B
K_E = EP // (NC * NS) // B
N_JUNK = 240
N_ACC = N_NODES + N_JUNK
ACC_ROWS_PER_TILE = N_ACC // NS

_EPS = 1e-5


def _sc_aggregate_x(x0, x1, src, dst):
    mesh = plsc.VectorSubcoreMesh(core_axis_name="c", subcore_axis_name="s")
    f32 = jnp.float32

    @functools.partial(
        pl.kernel,
        mesh=mesh,
        out_type=[
            jax.ShapeDtypeStruct((N_ACC, D_HALF), f32),
            jax.ShapeDtypeStruct((N_ACC, D_HALF), f32),
        ],
        scratch_types=[
            pltpu.VMEM((K_E, B), jnp.int32),
            pltpu.VMEM((K_E, B), jnp.int32),
            pltpu.VMEM((B, D_HALF), f32),
            pltpu.VMEM((B, D_HALF), f32),
            pltpu.VMEM_SHARED((N_ACC, D_HALF), f32),
            pltpu.SemaphoreType.DMA,
            pltpu.SemaphoreType.DMA,
        ],
    )
    def aggx_kernel(x0_hbm, x1_hbm, src_hbm, dst_hbm, outx0_hbm, outx1_hbm,
                    srcv, dstv, gbuf, gbuf1, accx, sem, sem1):
        c = lax.axis_index("c")
        s = lax.axis_index("s")

        zeros16 = jnp.zeros((16,), f32)

        @pl.loop(0, B)
        def _(r):
            @pl.loop(0, D_HALF, step=16)
            def _(q):
                gbuf[r, pl.ds(q, 16)] = zeros16

        @pl.loop(0, ACC_ROWS_PER_TILE, step=B)
        def _(r):
            pltpu.sync_copy(gbuf, accx.at[pl.ds(s * ACC_ROWS_PER_TILE + r, B)])

        plsc.subcore_barrier()

        def x_loop(xh_hbm):
            for p in range(2):
                pltpu.sync_copy(src_hbm.at[2 * s + p], srcv)
                pltpu.sync_copy(dst_hbm.at[2 * s + p], dstv)
                pltpu.async_copy(xh_hbm.at[srcv.at[0]], gbuf, sem)

                @pl.loop(0, K_E - 2, step=2)
                def _(j):
                    pltpu.make_async_copy(
                        xh_hbm.at[srcv.at[j]], gbuf, sem).wait()
                    pltpu.async_copy(xh_hbm.at[srcv.at[j + 1]], gbuf1, sem1)
                    pltpu.sync_copy(gbuf, accx.at[dstv.at[j]], add=True)
                    pltpu.make_async_copy(
                        xh_hbm.at[srcv.at[j + 1]], gbuf1, sem1).wait()
                    pltpu.async_copy(xh_hbm.at[srcv.at[j + 2]], gbuf, sem)
                    pltpu.sync_copy(gbuf1, accx.at[dstv.at[j + 1]], add=True)

                jt = K_E - 2
                pltpu.make_async_copy(xh_hbm.at[srcv.at[jt]], gbuf, sem).wait()
                pltpu.async_copy(xh_hbm.at[srcv.at[jt + 1]], gbuf1, sem1)
                pltpu.sync_copy(gbuf, accx.at[dstv.at[jt]], add=True)
                pltpu.make_async_copy(
                    xh_hbm.at[srcv.at[jt + 1]], gbuf1, sem1).wait()
                pltpu.sync_copy(gbuf1, accx.at[dstv.at[jt + 1]], add=True)

        @pl.when(c == 0)
        def _():
            x_loop(x0_hbm)

        @pl.when(c == 1)
        def _():
            x_loop(x1_hbm)

        plsc.subcore_barrier()

        rbase = s * ACC_ROWS_PER_TILE

        @pl.when(c == 0)
        def _():
            pltpu.sync_copy(accx.at[pl.ds(rbase, ACC_ROWS_PER_TILE)],
                            outx0_hbm.at[pl.ds(rbase, ACC_ROWS_PER_TILE)])

        @pl.when(c == 1)
        def _():
            pltpu.sync_copy(accx.at[pl.ds(rbase, ACC_ROWS_PER_TILE)],
                            outx1_hbm.at[pl.ds(rbase, ACC_ROWS_PER_TILE)])

    return aggx_kernel(x0, x1, src, dst)


def _sc_aggregate_e(dst, ea):
    mesh = plsc.VectorSubcoreMesh(core_axis_name="c", subcore_axis_name="s")
    f32 = jnp.float32

    @functools.partial(
        pl.kernel,
        mesh=mesh,
        out_type=jax.ShapeDtypeStruct((NC, N_ACC, EA_PAD), f32),
        scratch_types=[
            pltpu.VMEM((K_E, B), jnp.int32),
            pltpu.VMEM((D_EDGE, B), f32),
            pltpu.VMEM((D_EDGE, B), f32),
            pltpu.VMEM((B, EA_PAD), f32),
            pltpu.VMEM_SHARED((N_ACC, EA_PAD), f32),
            pltpu.SemaphoreType.DMA,
            pltpu.SemaphoreType.DMA,
        ],
    )
    def agge_kernel(dst_hbm, ea_hbm, oute_hbm,
                    dstev, rbuf0, rbuf1, ebuf0, acce, semr0, semr1):
        c = lax.axis_index("c")
        s = lax.axis_index("s")
        w = s * NC + c

        zeros16 = jnp.zeros((16,), f32)
        onehot = jnp.where(lax.iota(jnp.int32, 16) == 0,
                           jnp.float32(1), jnp.float32(0))

        @pl.loop(0, B)
        def _(r):
            @pl.loop(0, EA_PAD, step=16)
            def _(q):
                ebuf0[r, pl.ds(q, 16)] = zeros16

        @pl.loop(0, ACC_ROWS_PER_TILE, step=B)
        def _(r):
            pltpu.sync_copy(ebuf0, acce.at[pl.ds(s * ACC_ROWS_PER_TILE + r, B)])

        plsc.subcore_barrier()

        @pl.loop(0, B)
        def _(r):
            ebuf0[r, pl.ds(D_EDGE, 16)] = onehot

        pltpu.sync_copy(dst_hbm.at[w], dstev)

        def read(j, rb, semr):
            pltpu.async_copy(ea_hbm.at[w * K_E + j], rb, semr)

        def wait_read(j, rb, semr):
            pltpu.make_async_copy(ea_hbm.at[w * K_E + j], rb, semr).wait()

        def expand(rb):
            @pl.loop(0, D_EDGE)
            def _(rr):
                for k in range(8):
                    ebuf0[rr * 8 + k, pl.ds(0, D_EDGE)] = (
                        rb[rr, pl.ds(k * D_EDGE, D_EDGE)])

        def scat(j):
            pltpu.sync_copy(ebuf0, acce.at[dstev.at[j]], add=True)

        read(0, rbuf0, semr0)
        read(1, rbuf1, semr1)

        @pl.loop(0, K_E - 2, step=2)
        def _(j):
            wait_read(j, rbuf0, semr0)
            expand(rbuf0)
            read(j + 2, rbuf0, semr0)
            scat(j)
            wait_read(j + 1, rbuf1, semr1)
            expand(rbuf1)
            read(j + 3, rbuf1, semr1)
            scat(j + 1)

        jt = K_E - 2
        wait_read(jt, rbuf0, semr0)
        expand(rbuf0)
        scat(jt)
        wait_read(jt + 1, rbuf1, semr1)
        expand(rbuf1)
        scat(jt + 1)

        plsc.subcore_barrier()

        rbase = s * ACC_ROWS_PER_TILE
        pltpu.sync_copy(acce.at[pl.ds(rbase, ACC_ROWS_PER_TILE)],
                        oute_hbm.at[c, pl.ds(rbase, ACC_ROWS_PER_TILE)])

    return agge_kernel(dst, ea)


NB = 10
BLK = N_NODES // NB


def _dot(a, bm):
    return lax.dot_general(a, bm, (((1,), (0,)), ((), ())),
                           precision=lax.Precision.HIGHEST,
                           preferred_element_type=jnp.float32)


def _tc_a1_body(x_ref, ws_ref, b_ref, wr_ref,
                xw_ref, respre_ref, stats_ref):
    i = pl.program_id(0)
    xb = x_ref[...]
    r = _dot(xb, wr_ref[...])
    xw_ref[...] = _dot(xb, ws_ref[...]) + b_ref[...]
    respre_ref[...] = r

    st = jnp.concatenate(
        [jnp.sum(r, axis=0, keepdims=True),
         jnp.sum(r * r, axis=0, keepdims=True),
         jnp.zeros((6, D_OUT), jnp.float32)], axis=0)

    @pl.when(i == 0)
    def _():
        stats_ref[...] = st

    @pl.when(i > 0)
    def _():
        stats_ref[...] += st


def _tc_a1(x, ws, b2d, wr):
    blk = lambda cols: pl.BlockSpec((BLK, cols), lambda i: (i, 0))
    full = lambda a, c: pl.BlockSpec((a, c), lambda i: (0, 0))
    return pl.pallas_call(
        _tc_a1_body,
        grid=(NB,),
        in_specs=[blk(D_IN), full(D_IN, D_OUT), full(1, D_OUT),
                  full(D_IN, D_OUT)],
        out_specs=[blk(D_OUT), blk(D_OUT),
                   pl.BlockSpec((8, D_OUT), lambda i: (0, 0))],
        out_shape=[
            jax.ShapeDtypeStruct((N_NODES, D_OUT), jnp.float32),
            jax.ShapeDtypeStruct((N_NODES, D_OUT), jnp.float32),
            jax.ShapeDtypeStruct((8, D_OUT), jnp.float32),
        ],
    )(x, ws, b2d, wr)


def _tc_a2_body(xw_ref, ax0_ref, ax1_ref, ae_ref, wm_ref, we_ref,
                hpre_ref, stats_ref):
    i = pl.program_id(0)
    ae = ae_ref[0] + ae_ref[1]
    deg = jnp.maximum(ae[:, D_EDGE:D_EDGE + 1], 1.0)
    ssum = (_dot(ax0_ref[...], wm_ref[0:D_HALF, :])
            + _dot(ax1_ref[...], wm_ref[D_HALF:D_IN, :])
            + _dot(ae[:, 0:D_EDGE], we_ref[...]))
    h = ssum / deg + xw_ref[...]
    hpre_ref[...] = h

    st = jnp.concatenate(
        [jnp.sum(h, axis=0, keepdims=True),
         jnp.sum(h * h, axis=0, keepdims=True),
         jnp.zeros((6, D_OUT), jnp.float32)], axis=0)

    @pl.when(i == 0)
    def _():
        stats_ref[...] = st

    @pl.when(i > 0)
    def _():
        stats_ref[...] += st


def _tc_a2(xw, ax0, ax1, ae, wm, we):
    blk = lambda cols: pl.BlockSpec((BLK, cols), lambda i: (i, 0))
    full = lambda a, c: pl.BlockSpec((a, c), lambda i: (0, 0))
    return pl.pallas_call(
        _tc_a2_body,
        grid=(NB,),
        in_specs=[
            blk(D_OUT), blk(D_HALF), blk(D_HALF),
            pl.BlockSpec((NC, BLK, EA_PAD), lambda i: (0, i, 0)),
            full(D_IN, D_OUT), full(D_EDGE, D_OUT),
        ],
        out_specs=[blk(D_OUT),
                   pl.BlockSpec((8, D_OUT), lambda i: (0, 0))],
        out_shape=[
            jax.ShapeDtypeStruct((N_NODES, D_OUT), jnp.float32),
            jax.ShapeDtypeStruct((8, D_OUT), jnp.float32),
        ],
    )(xw, ax0, ax1, ae, wm, we)


def _tc_b_body(hpre_ref, respre_ref, stats_h_ref, stats_r_ref, g1_ref, b1_ref,
               g2_ref, b2_ref, a_ref, out_ref):
    sth = stats_h_ref[...]
    str_ = stats_r_ref[...]
    inv_n = 1.0 / N_NODES
    mu_h = sth[0:1, :] * inv_n
    var_h = sth[1:2, :] * inv_n - mu_h * mu_h
    mu_r = str_[0:1, :] * inv_n
    var_r = str_[1:2, :] * inv_n - mu_r * mu_r

    h = hpre_ref[...]
    hn = (h - mu_h) * lax.rsqrt(var_h + _EPS) * g1_ref[...] + b1_ref[...]
    a = a_ref[0, 0]
    hn = jnp.where(hn > 0, hn, a * hn)

    r = respre_ref[...]
    rn = (r - mu_r) * lax.rsqrt(var_r + _EPS) * g2_ref[...] + b2_ref[...]
    out_ref[...] = hn + rn


def _tc_b(hpre, respre, stats_h, stats_r, g1, b1, g2, b2, a2d):
    blk = pl.BlockSpec((BLK, D_OUT), lambda i: (i, 0))
    full = lambda r, c: pl.BlockSpec((r, c), lambda i: (0, 0))
    return pl.pallas_call(
        _tc_b_body,
        grid=(NB,),
        in_specs=[blk, blk, full(8, D_OUT), full(8, D_OUT), full(1, D_OUT),
                  full(1, D_OUT), full(1, D_OUT), full(1, D_OUT), full(1, 1)],
        out_specs=blk,
        out_shape=jax.ShapeDtypeStruct((N_NODES, D_OUT), jnp.float32),
    )(hpre, respre, stats_h, stats_r, g1, b1, g2, b2, a2d)


def kernel(x, edge_index, edge_attr, W_msg, W_edge, W_self, b,
           bn_gamma, bn_beta, prelu_a, W_res, rbn_gamma, rbn_beta):
    i32 = jnp.int32
    src = edge_index[0].astype(i32)
    dst = edge_index[1].astype(i32)

    npad = EP - N_EDGES
    pad_ar = jnp.arange(npad, dtype=i32)
    src_p = jnp.concatenate([src, pad_ar % N_NODES])
    dst_p = jnp.concatenate([dst, N_NODES + pad_ar % N_JUNK])

    src3 = src_p.reshape(NC * NS, K_E, B)
    dst3 = dst_p.reshape(NC * NS, K_E, B)

    ea = jnp.concatenate(
        [edge_attr, jnp.zeros((npad, D_EDGE), jnp.float32)], axis=0)
    ea = ea.reshape(NC * NS * K_E, D_EDGE, B)

    x0 = x[:, :D_HALF]
    x1 = x[:, D_HALF:]

    ax0, ax1 = _sc_aggregate_x(x0, x1, src3, dst3)
    ae = _sc_aggregate_e(dst3, ea)

    b2d = b.reshape(1, D_OUT)
    xw, respre, stats_r = _tc_a1(x, W_self, b2d, W_res)

    hpre, stats_h = _tc_a2(xw, ax0, ax1, ae, W_msg, W_edge)

    out = _tc_b(hpre, respre, stats_h, stats_r,
                bn_gamma.reshape(1, D_OUT), bn_beta.reshape(1, D_OUT),
                rbn_gamma.reshape(1, D_OUT), rbn_beta.reshape(1, D_OUT),
                jnp.asarray(prelu_a, jnp.float32).reshape(1, 1))
    return out


# speedup vs baseline: 5.0254x; 1.0014x over previous
import functools

import jax
import jax.numpy as jnp
from jax import lax
from jax.experimental import pallas as pl
from jax.experimental.pallas import tpu as pltpu
from jax.experimental.pallas import tpu_sc as plsc

N_NODES = 10000
N_EDGES = 160000
D_IN = 256
D_HALF = 128
D_OUT = 256
D_EDGE = 16
EA_PAD = 128

NC = 2
NS = 16
B = 128
EP = 163840
K_X = EP // NS // B
K_E = EP // (NC * NS) // B
N_JUNK = 240
N_ACC = N_NODES + N_JUNK
ACC_ROWS_PER_TILE = N_ACC // NS

_EPS = 1e-5


def _sc_aggregate_x(x0, x1, src, dst):
    mesh = plsc.VectorSubcoreMesh(core_axis_name="c", subcore_axis_name="s")
    f32 = jnp.float32

    @functools.partial(
        pl.kernel,
        mesh=mesh,
        out_type=[
            jax.ShapeDtypeStruct((N_ACC, D_HALF), f32),
            jax.ShapeDtypeStruct((N_ACC, D_HALF), f32),
        ],
        scratch_types=[
            pltpu.VMEM((K_E, B), jnp.int32),
            pltpu.VMEM((K_E, B), jnp.int32),
            pltpu.VMEM((B, D_HALF), f32),
            pltpu.VMEM((B, D_HALF), f32),
            pltpu.VMEM_SHARED((N_ACC, D_HALF), f32),
            pltpu.SemaphoreType.DMA,
            pltpu.SemaphoreType.DMA,
            pltpu.SemaphoreType.DMA,
            pltpu.SemaphoreType.DMA,
        ],
    )
    def aggx_kernel(x0_hbm, x1_hbm, src_hbm, dst_hbm, outx0_hbm, outx1_hbm,
                    srcv, dstv, gbuf, gbuf1, accx, sem, sem1, sems0, sems1):
        c = lax.axis_index("c")
        s = lax.axis_index("s")

        zeros16 = jnp.zeros((16,), f32)

        @pl.loop(0, B)
        def _(r):
            @pl.loop(0, D_HALF, step=16)
            def _(q):
                gbuf[r, pl.ds(q, 16)] = zeros16

        @pl.loop(0, ACC_ROWS_PER_TILE, step=B)
        def _(r):
            pltpu.sync_copy(gbuf, accx.at[pl.ds(s * ACC_ROWS_PER_TILE + r, B)])

        plsc.subcore_barrier()

        def wait_g(xh_hbm, j, gb, sm):
            pltpu.make_async_copy(xh_hbm.at[srcv.at[j]], gb, sm).wait()

        def scat(j, gb, sm):
            pltpu.async_copy(gb, accx.at[dstv.at[j]], sm, add=True)

        def wait_s(j, gb, sm):
            pltpu.make_async_copy(gb, accx.at[dstv.at[j]], sm).wait()

        def x_loop(xh_hbm):
            for p in range(2):
                pltpu.sync_copy(src_hbm.at[2 * s + p], srcv)
                pltpu.sync_copy(dst_hbm.at[2 * s + p], dstv)
                pltpu.async_copy(xh_hbm.at[srcv.at[0]], gbuf, sem)
                pltpu.async_copy(xh_hbm.at[srcv.at[1]], gbuf1, sem1)

                @pl.loop(0, K_E - 2, step=2)
                def _(j):
                    wait_g(xh_hbm, j, gbuf, sem)
                    scat(j, gbuf, sems0)
                    wait_g(xh_hbm, j + 1, gbuf1, sem1)
                    scat(j + 1, gbuf1, sems1)
                    wait_s(j, gbuf, sems0)
                    pltpu.async_copy(xh_hbm.at[srcv.at[j + 2]], gbuf, sem)
                    wait_s(j + 1, gbuf1, sems1)
                    pltpu.async_copy(xh_hbm.at[srcv.at[j + 3]], gbuf1, sem1)

                jt = K_E - 2
                wait_g(xh_hbm, jt, gbuf, sem)
                scat(jt, gbuf, sems0)
                wait_g(xh_hbm, jt + 1, gbuf1, sem1)
                scat(jt + 1, gbuf1, sems1)
                wait_s(jt, gbuf, sems0)
                wait_s(jt + 1, gbuf1, sems1)

        @pl.when(c == 0)
        def _():
            x_loop(x0_hbm)

        @pl.when(c == 1)
        def _():
            x_loop(x1_hbm)

        plsc.subcore_barrier()

        rbase = s * ACC_ROWS_PER_TILE

        @pl.when(c == 0)
        def _():
            pltpu.sync_copy(accx.at[pl.ds(rbase, ACC_ROWS_PER_TILE)],
                            outx0_hbm.at[pl.ds(rbase, ACC_ROWS_PER_TILE)])

        @pl.when(c == 1)
        def _():
            pltpu.sync_copy(accx.at[pl.ds(rbase, ACC_ROWS_PER_TILE)],
                            outx1_hbm.at[pl.ds(rbase, ACC_ROWS_PER_TILE)])

    return aggx_kernel(x0, x1, src, dst)


def _sc_aggregate_e(dst, ea):
    mesh = plsc.VectorSubcoreMesh(core_axis_name="c", subcore_axis_name="s")
    f32 = jnp.float32

    @functools.partial(
        pl.kernel,
        mesh=mesh,
        out_type=jax.ShapeDtypeStruct((NC, N_ACC, EA_PAD), f32),
        scratch_types=[
            pltpu.VMEM((K_E, B), jnp.int32),
            pltpu.VMEM((D_EDGE, B), f32),
            pltpu.VMEM((D_EDGE, B), f32),
            pltpu.VMEM((B, EA_PAD), f32),
            pltpu.VMEM_SHARED((N_ACC, EA_PAD), f32),
            pltpu.SemaphoreType.DMA,
            pltpu.SemaphoreType.DMA,
        ],
    )
    def agge_kernel(dst_hbm, ea_hbm, oute_hbm,
                    dstev, rbuf0, rbuf1, ebuf0, acce, semr0, semr1):
        c = lax.axis_index("c")
        s = lax.axis_index("s")
        w = s * NC + c

        zeros16 = jnp.zeros((16,), f32)
        onehot = jnp.where(lax.iota(jnp.int32, 16) == 0,
                           jnp.float32(1), jnp.float32(0))

        @pl.loop(0, B)
        def _(r):
            @pl.loop(0, EA_PAD, step=16)
            def _(q):
                ebuf0[r, pl.ds(q, 16)] = zeros16

        @pl.loop(0, ACC_ROWS_PER_TILE, step=B)
        def _(r):
            pltpu.sync_copy(ebuf0, acce.at[pl.ds(s * ACC_ROWS_PER_TILE + r, B)])

        plsc.subcore_barrier()

        @pl.loop(0, B)
        def _(r):
            ebuf0[r, pl.ds(D_EDGE, 16)] = onehot

        pltpu.sync_copy(dst_hbm.at[w], dstev)

        def read(j, rb, semr):
            pltpu.async_copy(ea_hbm.at[w * K_E + j], rb, semr)

        def wait_read(j, rb, semr):
            pltpu.make_async_copy(ea_hbm.at[w * K_E + j], rb, semr).wait()

        def expand(rb):
            @pl.loop(0, D_EDGE)
            def _(rr):
                for k in range(8):
                    ebuf0[rr * 8 + k, pl.ds(0, D_EDGE)] = (
                        rb[rr, pl.ds(k * D_EDGE, D_EDGE)])

        def scat(j):
            pltpu.sync_copy(ebuf0, acce.at[dstev.at[j]], add=True)

        read(0, rbuf0, semr0)
        read(1, rbuf1, semr1)

        @pl.loop(0, K_E - 2, step=2)
        def _(j):
            wait_read(j, rbuf0, semr0)
            expand(rbuf0)
            read(j + 2, rbuf0, semr0)
            scat(j)
            wait_read(j + 1, rbuf1, semr1)
            expand(rbuf1)
            read(j + 3, rbuf1, semr1)
            scat(j + 1)

        jt = K_E - 2
        wait_read(jt, rbuf0, semr0)
        expand(rbuf0)
        scat(jt)
        wait_read(jt + 1, rbuf1, semr1)
        expand(rbuf1)
        scat(jt + 1)

        plsc.subcore_barrier()

        rbase = s * ACC_ROWS_PER_TILE
        pltpu.sync_copy(acce.at[pl.ds(rbase, ACC_ROWS_PER_TILE)],
                        oute_hbm.at[c, pl.ds(rbase, ACC_ROWS_PER_TILE)])

    return agge_kernel(dst, ea)


NB = 10
BLK = N_NODES // NB


def _dot(a, bm):
    return lax.dot_general(a, bm, (((1,), (0,)), ((), ())),
                           precision=lax.Precision.HIGHEST,
                           preferred_element_type=jnp.float32)


def _tc_a1_body(x_ref, ws_ref, b_ref, wr_ref,
                xw_ref, respre_ref, stats_ref):
    i = pl.program_id(0)
    xb = x_ref[...]
    r = _dot(xb, wr_ref[...])
    xw_ref[...] = _dot(xb, ws_ref[...]) + b_ref[...]
    respre_ref[...] = r

    st = jnp.concatenate(
        [jnp.sum(r, axis=0, keepdims=True),
         jnp.sum(r * r, axis=0, keepdims=True),
         jnp.zeros((6, D_OUT), jnp.float32)], axis=0)

    @pl.when(i == 0)
    def _():
        stats_ref[...] = st

    @pl.when(i > 0)
    def _():
        stats_ref[...] += st


def _tc_a1(x, ws, b2d, wr):
    blk = lambda cols: pl.BlockSpec((BLK, cols), lambda i: (i, 0))
    full = lambda a, c: pl.BlockSpec((a, c), lambda i: (0, 0))
    return pl.pallas_call(
        _tc_a1_body,
        grid=(NB,),
        in_specs=[blk(D_IN), full(D_IN, D_OUT), full(1, D_OUT),
                  full(D_IN, D_OUT)],
        out_specs=[blk(D_OUT), blk(D_OUT),
                   pl.BlockSpec((8, D_OUT), lambda i: (0, 0))],
        out_shape=[
            jax.ShapeDtypeStruct((N_NODES, D_OUT), jnp.float32),
            jax.ShapeDtypeStruct((N_NODES, D_OUT), jnp.float32),
            jax.ShapeDtypeStruct((8, D_OUT), jnp.float32),
        ],
    )(x, ws, b2d, wr)


def _tc_a2_body(xw_ref, ax0_ref, ax1_ref, ae_ref, wm_ref, we_ref,
                hpre_ref, stats_ref):
    i = pl.program_id(0)
    ae = ae_ref[0] + ae_ref[1]
    deg = jnp.maximum(ae[:, D_EDGE:D_EDGE + 1], 1.0)
    ssum = (_dot(ax0_ref[...], wm_ref[0:D_HALF, :])
            + _dot(ax1_ref[...], wm_ref[D_HALF:D_IN, :])
            + _dot(ae[:, 0:D_EDGE], we_ref[...]))
    h = ssum / deg + xw_ref[...]
    hpre_ref[...] = h

    st = jnp.concatenate(
        [jnp.sum(h, axis=0, keepdims=True),
         jnp.sum(h * h, axis=0, keepdims=True),
         jnp.zeros((6, D_OUT), jnp.float32)], axis=0)

    @pl.when(i == 0)
    def _():
        stats_ref[...] = st

    @pl.when(i > 0)
    def _():
        stats_ref[...] += st


def _tc_a2(xw, ax0, ax1, ae, wm, we):
    blk = lambda cols: pl.BlockSpec((BLK, cols), lambda i: (i, 0))
    full = lambda a, c: pl.BlockSpec((a, c), lambda i: (0, 0))
    return pl.pallas_call(
        _tc_a2_body,
        grid=(NB,),
        in_specs=[
            blk(D_OUT), blk(D_HALF), blk(D_HALF),
            pl.BlockSpec((NC, BLK, EA_PAD), lambda i: (0, i, 0)),
            full(D_IN, D_OUT), full(D_EDGE, D_OUT),
        ],
        out_specs=[blk(D_OUT),
                   pl.BlockSpec((8, D_OUT), lambda i: (0, 0))],
        out_shape=[
            jax.ShapeDtypeStruct((N_NODES, D_OUT), jnp.float32),
            jax.ShapeDtypeStruct((8, D_OUT), jnp.float32),
        ],
    )(xw, ax0, ax1, ae, wm, we)


def _tc_b_body(hpre_ref, respre_ref, stats_h_ref, stats_r_ref, g1_ref, b1_ref,
               g2_ref, b2_ref, a_ref, out_ref):
    sth = stats_h_ref[...]
    str_ = stats_r_ref[...]
    inv_n = 1.0 / N_NODES
    mu_h = sth[0:1, :] * inv_n
    var_h = sth[1:2, :] * inv_n - mu_h * mu_h
    mu_r = str_[0:1, :] * inv_n
    var_r = str_[1:2, :] * inv_n - mu_r * mu_r

    h = hpre_ref[...]
    hn = (h - mu_h) * lax.rsqrt(var_h + _EPS) * g1_ref[...] + b1_ref[...]
    a = a_ref[0, 0]
    hn = jnp.where(hn > 0, hn, a * hn)

    r = respre_ref[...]
    rn = (r - mu_r) * lax.rsqrt(var_r + _EPS) * g2_ref[...] + b2_ref[...]
    out_ref[...] = hn + rn


def _tc_b(hpre, respre, stats_h, stats_r, g1, b1, g2, b2, a2d):
    blk = pl.BlockSpec((BLK, D_OUT), lambda i: (i, 0))
    full = lambda r, c: pl.BlockSpec((r, c), lambda i: (0, 0))
    return pl.pallas_call(
        _tc_b_body,
        grid=(NB,),
        in_specs=[blk, blk, full(8, D_OUT), full(8, D_OUT), full(1, D_OUT),
                  full(1, D_OUT), full(1, D_OUT), full(1, D_OUT), full(1, 1)],
        out_specs=blk,
        out_shape=jax.ShapeDtypeStruct((N_NODES, D_OUT), jnp.float32),
    )(hpre, respre, stats_h, stats_r, g1, b1, g2, b2, a2d)


def kernel(x, edge_index, edge_attr, W_msg, W_edge, W_self, b,
           bn_gamma, bn_beta, prelu_a, W_res, rbn_gamma, rbn_beta):
    i32 = jnp.int32
    src = edge_index[0].astype(i32)
    dst = edge_index[1].astype(i32)

    npad = EP - N_EDGES
    pad_ar = jnp.arange(npad, dtype=i32)
    src_p = jnp.concatenate([src, pad_ar % N_NODES])
    dst_p = jnp.concatenate([dst, N_NODES + pad_ar % N_JUNK])

    src3 = src_p.reshape(NC * NS, K_E, B)
    dst3 = dst_p.reshape(NC * NS, K_E, B)

    ea = jnp.concatenate(
        [edge_attr, jnp.zeros((npad, D_EDGE), jnp.float32)], axis=0)
    ea = ea.reshape(NC * NS * K_E, D_EDGE, B)

    x0 = x[:, :D_HALF]
    x1 = x[:, D_HALF:]

    ax0, ax1 = _sc_aggregate_x(x0, x1, src3, dst3)
    ae = _sc_aggregate_e(dst3, ea)

    b2d = b.reshape(1, D_OUT)
    xw, respre, stats_r = _tc_a1(x, W_self, b2d, W_res)

    hpre, stats_h = _tc_a2(xw, ax0, ax1, ae, W_msg, W_edge)

    out = _tc_b(hpre, respre, stats_h, stats_r,
                bn_gamma.reshape(1, D_OUT), bn_beta.reshape(1, D_OUT),
                rbn_gamma.reshape(1, D_OUT), rbn_beta.reshape(1, D_OUT),
                jnp.asarray(prelu_a, jnp.float32).reshape(1, 1))
    return out
